# baseline (speedup 1.0000x reference)
import functools

import jax
import jax.numpy as jnp
from jax import lax
from jax.experimental import pallas as pl
from jax.experimental.pallas import tpu as pltpu
from jax.experimental.pallas import tpu_sc as plsc

N = 4194304
NUM_CORES = 2
NUM_SUBCORES = 16
NUM_WORKERS = NUM_CORES * NUM_SUBCORES
TOKENS_PER_WORKER = N // NUM_WORKERS
W = 64
NROWS = N // W
CHUNK = 8192
CROWS = CHUNK // W
ROWS_PER_WORKER = TOKENS_PER_WORKER // W
NUM_CHUNKS = TOKENS_PER_WORKER // CHUNK
LANES = 16
IN_SPLIT = 4

_mesh = plsc.VectorSubcoreMesh(core_axis_name="c", subcore_axis_name="s")


@functools.partial(
    pl.kernel,
    mesh=_mesh,
    out_type=jax.ShapeDtypeStruct((4, N), jnp.float32),
    scratch_types=[
        pltpu.VMEM((2, CROWS, W), jnp.int32),
        pltpu.VMEM((2, 4, CHUNK), jnp.float32),
        pltpu.SemaphoreType.DMA,
        pltpu.SemaphoreType.DMA,
        pltpu.SemaphoreType.DMA,
        pltpu.SemaphoreType.DMA,
    ],
)
def _seq2tensor_sc(seq_hbm, out_hbm, seq_v, out_v, in_sem0, in_sem1,
                   out_sem0, out_sem1):
    in_sems = (in_sem0, in_sem1)
    out_sems = (out_sem0, out_sem1)
    wid = lax.axis_index("s") * NUM_CORES + lax.axis_index("c")
    base = wid * TOKENS_PER_WORKER
    rbase = wid * ROWS_PER_WORKER

    def start_in(ci, b):
        r = CROWS // IN_SPLIT
        for k in range(IN_SPLIT):
            pltpu.async_copy(
                seq_hbm.at[pl.ds(rbase + ci * CROWS + k * r, r)],
                seq_v.at[b, pl.ds(k * r, r)],
                in_sems[b])

    def wait_in(b):
        pltpu.make_async_copy(
            seq_hbm.at[pl.ds(rbase, CROWS)], seq_v.at[b], in_sems[b]).wait()

    def start_out(ci, b):
        @plsc.parallel_loop(0, CROWS, step=1, unroll=4)
        def _(r):
            off = r * W
            for c in range(4):
                pltpu.async_copy(
                    out_v.at[b, c, pl.ds(off, W)],
                    out_hbm.at[c, pl.ds(base + ci * CHUNK + off, W)],
                    out_sems[b])

    def wait_out(b):
        for c in range(4):
            pltpu.make_async_copy(
                out_v.at[b, c], out_hbm.at[c, pl.ds(base, CHUNK)],
                out_sems[b]).wait()

    lane = lax.iota(jnp.int32, LANES)
    luts = [
        jnp.where(lane == c, 1.0,
                  jnp.where(lane == 4, 0.25, 0.0)).astype(jnp.float32)
        for c in range(4)
    ]

    _dnums = lax.GatherDimensionNumbers(
        offset_dims=(), collapsed_slice_dims=(0,), start_index_map=(0,))

    def _lut_lookup(lut, s):
        return lax.gather(lut, s[:, None], _dnums, slice_sizes=(1,),
                          mode=lax.GatherScatterMode.PROMISE_IN_BOUNDS)

    def compute(b):
        @plsc.parallel_loop(0, CROWS, step=1, unroll=2)
        def _(r):
            off = r * W
            for j in range(W // LANES):
                s = seq_v[b, r, pl.ds(j * LANES, LANES)]
                for c in range(4):
                    out_v[b, c, pl.ds(off + j * LANES, LANES)] = _lut_lookup(
                        luts[c], s)

    start_in(0, 0)
    for ci in range(NUM_CHUNKS):
        b = ci % 2
        wait_in(b)
        if ci + 1 < NUM_CHUNKS:
            start_in(ci + 1, 1 - b)
        if ci >= 2:
            wait_out(b)
        compute(b)
        start_out(ci, b)
    wait_out(0)
    wait_out(1)


def kernel(seq):
    return _seq2tensor_sc(seq.reshape(NROWS, W))


# speedup vs baseline: 1.2807x; 1.2807x over previous
import functools

import jax
import jax.numpy as jnp
from jax import lax
from jax.experimental import pallas as pl
from jax.experimental.pallas import tpu as pltpu
from jax.experimental.pallas import tpu_sc as plsc

N = 4194304
NUM_CORES = 2
NUM_SUBCORES = 16
NUM_WORKERS = NUM_CORES * NUM_SUBCORES
TOKENS_PER_WORKER = N // NUM_WORKERS
W = 128
NROWS = N // W
CHUNK = 8192
CROWS = CHUNK // W
ROWS_PER_WORKER = TOKENS_PER_WORKER // W
NUM_CHUNKS = TOKENS_PER_WORKER // CHUNK
LANES = 16
IN_SPLIT = 4

_mesh = plsc.VectorSubcoreMesh(core_axis_name="c", subcore_axis_name="s")


@functools.partial(
    pl.kernel,
    mesh=_mesh,
    out_type=jax.ShapeDtypeStruct((4, N), jnp.float32),
    scratch_types=[
        pltpu.VMEM((2, CROWS, W), jnp.int32),
        pltpu.VMEM((2, 4, CHUNK), jnp.float32),
        pltpu.SemaphoreType.DMA,
        pltpu.SemaphoreType.DMA,
        pltpu.SemaphoreType.DMA,
        pltpu.SemaphoreType.DMA,
    ],
)
def _seq2tensor_sc(seq_hbm, out_hbm, seq_v, out_v, in_sem0, in_sem1,
                   out_sem0, out_sem1):
    in_sems = (in_sem0, in_sem1)
    out_sems = (out_sem0, out_sem1)
    wid = lax.axis_index("s") * NUM_CORES + lax.axis_index("c")
    base = wid * TOKENS_PER_WORKER
    rbase = wid * ROWS_PER_WORKER

    def start_in(ci, b):
        r = CROWS // IN_SPLIT
        for k in range(IN_SPLIT):
            pltpu.async_copy(
                seq_hbm.at[pl.ds(rbase + ci * CROWS + k * r, r)],
                seq_v.at[b, pl.ds(k * r, r)],
                in_sems[b])

    def wait_in(b):
        pltpu.make_async_copy(
            seq_hbm.at[pl.ds(rbase, CROWS)], seq_v.at[b], in_sems[b]).wait()

    def start_out(ci, b):
        @plsc.parallel_loop(0, CROWS, step=1, unroll=4)
        def _(r):
            off = r * W
            for c in range(4):
                pltpu.async_copy(
                    out_v.at[b, c, pl.ds(off, W)],
                    out_hbm.at[c, pl.ds(base + ci * CHUNK + off, W)],
                    out_sems[b])

    def wait_out(b):
        for c in range(4):
            pltpu.make_async_copy(
                out_v.at[b, c], out_hbm.at[c, pl.ds(base, CHUNK)],
                out_sems[b]).wait()

    lane = lax.iota(jnp.int32, LANES)
    luts = [
        jnp.where(lane == c, 1.0,
                  jnp.where(lane == 4, 0.25, 0.0)).astype(jnp.float32)
        for c in range(4)
    ]

    _dnums = lax.GatherDimensionNumbers(
        offset_dims=(), collapsed_slice_dims=(0,), start_index_map=(0,))

    def _lut_lookup(lut, s):
        return lax.gather(lut, s[:, None], _dnums, slice_sizes=(1,),
                          mode=lax.GatherScatterMode.PROMISE_IN_BOUNDS)

    def compute(b):
        @plsc.parallel_loop(0, CROWS, step=1, unroll=2)
        def _(r):
            off = r * W
            for j in range(W // LANES):
                s = seq_v[b, r, pl.ds(j * LANES, LANES)]
                for c in range(4):
                    out_v[b, c, pl.ds(off + j * LANES, LANES)] = _lut_lookup(
                        luts[c], s)

    start_in(0, 0)
    for ci in range(NUM_CHUNKS):
        b = ci % 2
        wait_in(b)
        if ci + 1 < NUM_CHUNKS:
            start_in(ci + 1, 1 - b)
        if ci >= 2:
            wait_out(b)
        compute(b)
        start_out(ci, b)
    wait_out(0)
    wait_out(1)


def kernel(seq):
    return _seq2tensor_sc(seq.reshape(NROWS, W))
